# baseline (speedup 1.0000x reference)
import functools

import jax
import jax.numpy as jnp
from jax import lax
from jax.experimental import pallas as pl
from jax.experimental.pallas import tpu as pltpu
from jax.experimental.pallas import tpu_sc as plsc

B = 4096
L = 26
D = 64
V = 100000

NC = 2
NS = 16
NW = NC * NS
BPW = B // NW
NB = 16
NCHUNK = BPW // NB
ROWS = NB * L
NSUB = 4
SUB = ROWS // NSUB
BLK = D * L
NVREG = BLK // 16


def _sc_body(x_hbm, table_hbm, out_hbm, idx_v, rows_v, tout_v, gsem):
    wid = lax.axis_index("s") * NC + lax.axis_index("c")
    base_b = wid * BPW

    def chunk_body(ci, carry):
        b0 = base_b + ci * NB
        for j in range(NSUB):
            pltpu.sync_copy(x_hbm.at[pl.ds(b0 * L + j * SUB, SUB)], idx_v.at[j])
        cps = [
            pltpu.async_copy(
                table_hbm.at[idx_v.at[j]],
                rows_v.at[pl.ds(j * SUB, SUB)],
                gsem,
            )
            for j in range(NSUB)
        ]
        for cp in cps:
            cp.wait()

        stride = lax.iota(jnp.int32, 16) * L

        def r_body(r, _):
            kk = lax.div(r, L)
            l = r - kk * L
            base = kk * BLK + l
            for c0 in range(0, D, 16):
                vals = rows_v[r, pl.ds(c0, 16)]
                plsc.store_scatter(tout_v, [stride + (base + c0 * L)], vals)
            return 0

        lax.fori_loop(0, ROWS, r_body, 0, unroll=False)

        pltpu.sync_copy(tout_v, out_hbm.at[pl.ds(b0 * BLK, NB * BLK)])
        return carry

    lax.fori_loop(0, NCHUNK, chunk_body, 0, unroll=False)


@functools.partial(jax.jit, static_argnames=())
def kernel(x, table):
    x_flat = x.reshape(-1).astype(jnp.int32)
    mesh = plsc.VectorSubcoreMesh(core_axis_name="c", subcore_axis_name="s")
    out_flat = pl.kernel(
        _sc_body,
        out_type=jax.ShapeDtypeStruct((B * D * L,), jnp.float32),
        mesh=mesh,
        compiler_params=pltpu.CompilerParams(needs_layout_passes=False, use_tc_tiling_on_sc=False),
        scratch_types=[
            pltpu.VMEM((NSUB, SUB), jnp.int32),
            pltpu.VMEM((ROWS, D), jnp.float32),
            pltpu.VMEM((NB * BLK,), jnp.float32),
            pltpu.SemaphoreType.DMA,
        ],
    )(x_flat, table)
    return out_flat.reshape(B, D, L)


# speedup vs baseline: 2.5350x; 2.5350x over previous
import functools

import jax
import jax.numpy as jnp
from jax import lax
from jax.experimental import pallas as pl
from jax.experimental.pallas import tpu as pltpu
from jax.experimental.pallas import tpu_sc as plsc

B = 4096
L = 26
D = 64
V = 100000

NC = 2
NS = 16
NW = NC * NS
CPW = D // NW
NJ = B // 16


def _sc_body(xT_hbm, tT_hbm, outT_hbm, row_v, idx_v, outb_v):
    wid = lax.axis_index("s") * NC + lax.axis_index("c")

    for ci in range(CPW):
        c = wid * CPW + ci
        pltpu.sync_copy(tT_hbm.at[c, pl.ds(0, V)], row_v)

        def l_body(l, _):
            pltpu.sync_copy(xT_hbm.at[l, pl.ds(0, B)], idx_v)

            def j_body(j, _):
                for u in range(8):
                    off = (j * 8 + u) * 16
                    idx = idx_v[pl.ds(off, 16)]
                    outb_v[pl.ds(off, 16)] = plsc.load_gather(row_v, [idx])
                return 0

            lax.fori_loop(0, NJ // 8, j_body, 0)
            pltpu.sync_copy(outb_v, outT_hbm.at[l, c, pl.ds(0, B)])
            return 0

        lax.fori_loop(0, L, l_body, 0)


@jax.jit
def kernel(x, table):
    xT = x.T.astype(jnp.int32)
    tT = table.T
    mesh = plsc.VectorSubcoreMesh(core_axis_name="c", subcore_axis_name="s")
    outT = pl.kernel(
        _sc_body,
        out_type=jax.ShapeDtypeStruct((L, D, B), jnp.float32),
        mesh=mesh,
        compiler_params=pltpu.CompilerParams(
            needs_layout_passes=False, use_tc_tiling_on_sc=True
        ),
        scratch_types=[
            pltpu.VMEM((V,), jnp.float32),
            pltpu.VMEM((B,), jnp.int32),
            pltpu.VMEM((B,), jnp.float32),
        ],
    )(xT, tT)
    return jnp.transpose(outT, (2, 1, 0))


# speedup vs baseline: 3.1964x; 1.2609x over previous
import functools

import jax
import jax.numpy as jnp
from jax import lax
from jax.experimental import pallas as pl
from jax.experimental.pallas import tpu as pltpu
from jax.experimental.pallas import tpu_sc as plsc

B = 4096
L = 26
D = 64
V = 100000

NC = 2
NS = 16
NW = NC * NS
CPW = D // NW
NJ = B // 16


def _sc_body(xT_hbm, tT_hbm, outT_hbm, row_v, idx_v, outb_v,
             sem_i0, sem_i1, sem_o0, sem_o1):
    wid = lax.axis_index("s") * NC + lax.axis_index("c")
    sem_i = (sem_i0, sem_i1)
    sem_o = (sem_o0, sem_o1)

    for ci in range(CPW):
        c = wid * CPW + ci
        pltpu.sync_copy(tT_hbm.at[c, pl.ds(0, V)], row_v)

        idx_cp = [None] * L
        out_cp = [None] * L
        idx_cp[0] = pltpu.async_copy(
            xT_hbm.at[0, pl.ds(0, B)], idx_v.at[0], sem_i[0])
        for l in range(L):
            par = l % 2
            if l + 1 < L:
                idx_cp[l + 1] = pltpu.async_copy(
                    xT_hbm.at[l + 1, pl.ds(0, B)], idx_v.at[1 - par],
                    sem_i[1 - par])
            idx_cp[l].wait()
            if l >= 2:
                out_cp[l - 2].wait()

            def j_body(j, _):
                for u in range(8):
                    off = (j * 8 + u) * 16
                    idx = idx_v[par, pl.ds(off, 16)]
                    outb_v[par, pl.ds(off, 16)] = plsc.load_gather(
                        row_v, [idx])
                return 0

            lax.fori_loop(0, NJ // 8, j_body, 0)
            out_cp[l] = pltpu.async_copy(
                outb_v.at[par], outT_hbm.at[l, c, pl.ds(0, B)], sem_o[par])
        out_cp[L - 2].wait()
        out_cp[L - 1].wait()


@jax.jit
def kernel(x, table):
    xT = x.T.astype(jnp.int32)
    tT = table.T
    mesh = plsc.VectorSubcoreMesh(core_axis_name="c", subcore_axis_name="s")
    outT = pl.kernel(
        _sc_body,
        out_type=jax.ShapeDtypeStruct((L, D, B), jnp.float32),
        mesh=mesh,
        compiler_params=pltpu.CompilerParams(
            needs_layout_passes=False, use_tc_tiling_on_sc=True
        ),
        scratch_types=[
            pltpu.VMEM((V,), jnp.float32),
            pltpu.VMEM((2, B), jnp.int32),
            pltpu.VMEM((2, B), jnp.float32),
            pltpu.SemaphoreType.DMA,
            pltpu.SemaphoreType.DMA,
            pltpu.SemaphoreType.DMA,
            pltpu.SemaphoreType.DMA,
        ],
    )(xT, tT)
    return jnp.transpose(outT, (2, 1, 0))


# speedup vs baseline: 4.1786x; 1.3073x over previous
import functools

import jax
import jax.numpy as jnp
from jax import lax
from jax.experimental import pallas as pl
from jax.experimental.pallas import tpu as pltpu
from jax.experimental.pallas import tpu_sc as plsc

B = 4096
L = 26
D = 64
V = 100000

NC = 2
NS = 16
NW = NC * NS
CPW = D // NW
NJ = B // 16


def _sc_body(xT_hbm, tT_hbm, outT_hbm, row_v, idx_v, outb_v,
             sem_i0, sem_i1, sem_o0, sem_o1):
    wid = lax.axis_index("s") * NC + lax.axis_index("c")
    sem_i = (sem_i0, sem_i1)
    sem_o = (sem_o0, sem_o1)

    for ci in range(CPW):
        c = wid * CPW + ci
        pltpu.sync_copy(tT_hbm.at[c, pl.ds(0, V)], row_v)

        idx_cp = [None] * L
        out_cp = [None] * L
        idx_cp[0] = pltpu.async_copy(
            xT_hbm.at[0, pl.ds(0, B)], idx_v.at[0], sem_i[0])
        for l in range(L):
            par = l % 2
            if l + 1 < L:
                idx_cp[l + 1] = pltpu.async_copy(
                    xT_hbm.at[l + 1, pl.ds(0, B)], idx_v.at[1 - par],
                    sem_i[1 - par])
            idx_cp[l].wait()
            if l >= 2:
                out_cp[l - 2].wait()

            @plsc.parallel_loop(0, B, step=16, unroll=8)
            def _gather(off):
                idx = idx_v[par, pl.ds(off, 16)]
                outb_v[par, pl.ds(off, 16)] = plsc.load_gather(row_v, [idx])
            out_cp[l] = pltpu.async_copy(
                outb_v.at[par], outT_hbm.at[l, c, pl.ds(0, B)], sem_o[par])
        out_cp[L - 2].wait()
        out_cp[L - 1].wait()


@jax.jit
def kernel(x, table):
    xT = x.T.astype(jnp.int32)
    tT = table.T
    mesh = plsc.VectorSubcoreMesh(core_axis_name="c", subcore_axis_name="s")
    outT = pl.kernel(
        _sc_body,
        out_type=jax.ShapeDtypeStruct((L, D, B), jnp.float32),
        mesh=mesh,
        compiler_params=pltpu.CompilerParams(
            needs_layout_passes=False, use_tc_tiling_on_sc=True
        ),
        scratch_types=[
            pltpu.VMEM((V,), jnp.float32),
            pltpu.VMEM((2, B), jnp.int32),
            pltpu.VMEM((2, B), jnp.float32),
            pltpu.SemaphoreType.DMA,
            pltpu.SemaphoreType.DMA,
            pltpu.SemaphoreType.DMA,
            pltpu.SemaphoreType.DMA,
        ],
    )(xT, tT)
    return jnp.transpose(outT, (2, 1, 0))


# speedup vs baseline: 4.1889x; 1.0025x over previous
import functools

import jax
import jax.numpy as jnp
from jax import lax
from jax.experimental import pallas as pl
from jax.experimental.pallas import tpu as pltpu
from jax.experimental.pallas import tpu_sc as plsc

B = 4096
L = 26
D = 64
V = 100000

NC = 2
NS = 16
NW = NC * NS
CPW = D // NW
NJ = B // 16


def _sc_body(xT_hbm, tT_hbm, outT_hbm, row_v, idx_v, outb_v,
             sem_i0, sem_i1, sem_o0, sem_o1):
    wid = lax.axis_index("s") * NC + lax.axis_index("c")
    sem_i = (sem_i0, sem_i1)
    sem_o = (sem_o0, sem_o1)

    for ci in range(CPW):
        c = wid * CPW + ci
        with jax.named_scope("row_dma"):
            pltpu.sync_copy(tT_hbm.at[c, pl.ds(0, V)], row_v)

        idx_cp = [None] * L
        out_cp = [None] * L
        idx_cp[0] = pltpu.async_copy(
            xT_hbm.at[0, pl.ds(0, B)], idx_v.at[0], sem_i[0])
        for l in range(L):
            par = l % 2
            if l + 1 < L:
                idx_cp[l + 1] = pltpu.async_copy(
                    xT_hbm.at[l + 1, pl.ds(0, B)], idx_v.at[1 - par],
                    sem_i[1 - par])
            with jax.named_scope("dma_waits"):
                idx_cp[l].wait()
                if l >= 2:
                    out_cp[l - 2].wait()

            with jax.named_scope("gather"):
                @plsc.parallel_loop(0, B, step=16, unroll=8)
                def _gather(off):
                    idx = idx_v[par, pl.ds(off, 16)]
                    outb_v[par, pl.ds(off, 16)] = plsc.load_gather(
                        row_v, [idx])
            out_cp[l] = pltpu.async_copy(
                outb_v.at[par], outT_hbm.at[l, c, pl.ds(0, B)], sem_o[par])
        out_cp[L - 2].wait()
        out_cp[L - 1].wait()


@jax.jit
def kernel(x, table):
    xT = x.T.astype(jnp.int32)
    tT = table.T
    mesh = plsc.VectorSubcoreMesh(core_axis_name="c", subcore_axis_name="s")
    outT = pl.kernel(
        _sc_body,
        out_type=jax.ShapeDtypeStruct((L, D, B), jnp.float32),
        mesh=mesh,
        compiler_params=pltpu.CompilerParams(
            needs_layout_passes=False, use_tc_tiling_on_sc=True
        ),
        scratch_types=[
            pltpu.VMEM((V,), jnp.float32),
            pltpu.VMEM((2, B), jnp.int32),
            pltpu.VMEM((2, B), jnp.float32),
            pltpu.SemaphoreType.DMA,
            pltpu.SemaphoreType.DMA,
            pltpu.SemaphoreType.DMA,
            pltpu.SemaphoreType.DMA,
        ],
    )(xT, tT)
    return jnp.transpose(outT, (2, 1, 0))


# speedup vs baseline: 4.8744x; 1.1637x over previous
import functools

import jax
import jax.numpy as jnp
from jax import lax
from jax.experimental import pallas as pl
from jax.experimental.pallas import tpu as pltpu
from jax.experimental.pallas import tpu_sc as plsc

B = 4096
L = 26
D = 64
V = 100000

NC = 2
NS = 16
NW = NC * NS
CPW = D // NW
NJ = B // 16
NBUF = 3


def _sc_body(xT_hbm, tT_hbm, outT_hbm, row_v, idx0_v, idx1_v, idx2_v,
             outb0_v, outb1_v, outb2_v,
             sem_i0, sem_i1, sem_i2, sem_o0, sem_o1, sem_o2):
    wid = lax.axis_index("s") * NC + lax.axis_index("c")
    idx_v = (idx0_v, idx1_v, idx2_v)
    outb_v = (outb0_v, outb1_v, outb2_v)
    sem_i = (sem_i0, sem_i1, sem_i2)
    sem_o = (sem_o0, sem_o1, sem_o2)

    for ci in range(CPW):
        c = wid * CPW + ci
        idx_cp = [None] * L
        out_cp = [None] * L
        for p in range(NBUF):
            idx_cp[p] = pltpu.async_copy(
                xT_hbm.at[p, pl.ds(0, B)], idx_v[p], sem_i[p])
        pltpu.sync_copy(tT_hbm.at[c, pl.ds(0, V)], row_v)
        for l in range(L):
            par = l % NBUF
            idx_cp[l].wait()
            if l >= NBUF:
                out_cp[l - NBUF].wait()

            @plsc.parallel_loop(0, B, step=16, unroll=8)
            def _gather(off):
                idx = idx_v[par][pl.ds(off, 16)]
                outb_v[par][pl.ds(off, 16)] = plsc.load_gather(row_v, [idx])

            out_cp[l] = pltpu.async_copy(
                outb_v[par], outT_hbm.at[l, c, pl.ds(0, B)], sem_o[par])
            if l + NBUF < L:
                idx_cp[l + NBUF] = pltpu.async_copy(
                    xT_hbm.at[l + NBUF, pl.ds(0, B)], idx_v[par],
                    sem_i[par])
        for t in range(NBUF):
            out_cp[L - NBUF + t].wait()


@jax.jit
def kernel(x, table):
    xT = x.T.astype(jnp.int32)
    tT = table.T
    mesh = plsc.VectorSubcoreMesh(core_axis_name="c", subcore_axis_name="s")
    outT = pl.kernel(
        _sc_body,
        out_type=jax.ShapeDtypeStruct((L, D, B), jnp.float32),
        mesh=mesh,
        compiler_params=pltpu.CompilerParams(
            needs_layout_passes=False, use_tc_tiling_on_sc=True
        ),
        scratch_types=[
            pltpu.VMEM((V,), jnp.float32),
            pltpu.VMEM((B,), jnp.int32),
            pltpu.VMEM((B,), jnp.int32),
            pltpu.VMEM((B,), jnp.int32),
            pltpu.VMEM((B,), jnp.float32),
            pltpu.VMEM((B,), jnp.float32),
            pltpu.VMEM((B,), jnp.float32),
            pltpu.SemaphoreType.DMA,
            pltpu.SemaphoreType.DMA,
            pltpu.SemaphoreType.DMA,
            pltpu.SemaphoreType.DMA,
            pltpu.SemaphoreType.DMA,
            pltpu.SemaphoreType.DMA,
        ],
    )(xT, tT)
    return jnp.transpose(outT, (2, 1, 0))


# speedup vs baseline: 4.9515x; 1.0158x over previous
import functools

import jax
import jax.numpy as jnp
from jax import lax
from jax.experimental import pallas as pl
from jax.experimental.pallas import tpu as pltpu
from jax.experimental.pallas import tpu_sc as plsc

B = 4096
L = 26
D = 64
V = 100000

NC = 2
NS = 16
NW = NC * NS
CPW = D // NW
NJ = B // 16
NBI = 4
NBO = 3


def _sc_body(xT_hbm, tT_hbm, outT_hbm, row_v, idx0_v, idx1_v, idx2_v,
             idx3_v, outb0_v, outb1_v, outb2_v,
             sem_i0, sem_i1, sem_i2, sem_i3, sem_o0, sem_o1, sem_o2):
    wid = lax.axis_index("s") * NC + lax.axis_index("c")
    idx_v = (idx0_v, idx1_v, idx2_v, idx3_v)
    outb_v = (outb0_v, outb1_v, outb2_v)
    sem_i = (sem_i0, sem_i1, sem_i2, sem_i3)
    sem_o = (sem_o0, sem_o1, sem_o2)

    for ci in range(CPW):
        c = wid * CPW + ci
        idx_cp = [None] * L
        out_cp = [None] * L
        for p in range(NBI):
            idx_cp[p] = pltpu.async_copy(
                xT_hbm.at[p, pl.ds(0, B)], idx_v[p], sem_i[p])
        pltpu.sync_copy(tT_hbm.at[c, pl.ds(0, V)], row_v)
        for l in range(L):
            pi = l % NBI
            po = l % NBO
            idx_cp[l].wait()
            if l >= NBO:
                out_cp[l - NBO].wait()

            @plsc.parallel_loop(0, B, step=16, unroll=8)
            def _gather(off):
                idx = idx_v[pi][pl.ds(off, 16)]
                outb_v[po][pl.ds(off, 16)] = plsc.load_gather(row_v, [idx])

            out_cp[l] = pltpu.async_copy(
                outb_v[po], outT_hbm.at[l, c, pl.ds(0, B)], sem_o[po])
            if l + NBI < L:
                idx_cp[l + NBI] = pltpu.async_copy(
                    xT_hbm.at[l + NBI, pl.ds(0, B)], idx_v[pi],
                    sem_i[pi])
        for t in range(NBO):
            out_cp[L - NBO + t].wait()


@jax.jit
def kernel(x, table):
    xT = x.T.astype(jnp.int32)
    tT = table.T
    mesh = plsc.VectorSubcoreMesh(core_axis_name="c", subcore_axis_name="s")
    outT = pl.kernel(
        _sc_body,
        out_type=jax.ShapeDtypeStruct((L, D, B), jnp.float32),
        mesh=mesh,
        compiler_params=pltpu.CompilerParams(
            needs_layout_passes=False, use_tc_tiling_on_sc=True
        ),
        scratch_types=[
            pltpu.VMEM((V,), jnp.float32),
            pltpu.VMEM((B,), jnp.int32),
            pltpu.VMEM((B,), jnp.int32),
            pltpu.VMEM((B,), jnp.int32),
            pltpu.VMEM((B,), jnp.int32),
            pltpu.VMEM((B,), jnp.float32),
            pltpu.VMEM((B,), jnp.float32),
            pltpu.VMEM((B,), jnp.float32),
            pltpu.SemaphoreType.DMA,
            pltpu.SemaphoreType.DMA,
            pltpu.SemaphoreType.DMA,
            pltpu.SemaphoreType.DMA,
            pltpu.SemaphoreType.DMA,
            pltpu.SemaphoreType.DMA,
            pltpu.SemaphoreType.DMA,
        ],
    )(xT, tT)
    return jnp.transpose(outT, (2, 1, 0))


# speedup vs baseline: 5.0508x; 1.0201x over previous
import functools

import jax
import jax.numpy as jnp
from jax import lax
from jax.experimental import pallas as pl
from jax.experimental.pallas import tpu as pltpu
from jax.experimental.pallas import tpu_sc as plsc

B = 4096
L = 26
D = 64
V = 100000

NC = 2
NS = 16
NW = NC * NS
CPW = D // NW
NJ = B // 16
NBI = 5
NBO = 2


def _sc_body(xT_hbm, tT_hbm, outT_hbm, row_v, idx0_v, idx1_v, idx2_v,
             idx3_v, idx4_v, outb0_v, outb1_v,
             sem_i0, sem_i1, sem_i2, sem_i3, sem_i4, sem_o0, sem_o1):
    wid = lax.axis_index("s") * NC + lax.axis_index("c")
    idx_v = (idx0_v, idx1_v, idx2_v, idx3_v, idx4_v)
    outb_v = (outb0_v, outb1_v)
    sem_i = (sem_i0, sem_i1, sem_i2, sem_i3, sem_i4)
    sem_o = (sem_o0, sem_o1)

    for ci in range(CPW):
        c = wid * CPW + ci
        idx_cp = [None] * L
        out_cp = [None] * L
        for p in range(NBI):
            idx_cp[p] = pltpu.async_copy(
                xT_hbm.at[p, pl.ds(0, B)], idx_v[p], sem_i[p])
        pltpu.sync_copy(tT_hbm.at[c, pl.ds(0, V)], row_v)
        for l in range(L):
            pi = l % NBI
            po = l % NBO
            idx_cp[l].wait()
            if l >= NBO:
                out_cp[l - NBO].wait()

            @plsc.parallel_loop(0, B, step=16, unroll=8)
            def _gather(off):
                idx = idx_v[pi][pl.ds(off, 16)]
                outb_v[po][pl.ds(off, 16)] = plsc.load_gather(row_v, [idx])

            out_cp[l] = pltpu.async_copy(
                outb_v[po], outT_hbm.at[l, c, pl.ds(0, B)], sem_o[po])
            if l + NBI < L:
                idx_cp[l + NBI] = pltpu.async_copy(
                    xT_hbm.at[l + NBI, pl.ds(0, B)], idx_v[pi],
                    sem_i[pi])
        for t in range(NBO):
            out_cp[L - NBO + t].wait()


@jax.jit
def kernel(x, table):
    xT = x.T.astype(jnp.int32)
    tT = table.T
    mesh = plsc.VectorSubcoreMesh(core_axis_name="c", subcore_axis_name="s")
    outT = pl.kernel(
        _sc_body,
        out_type=jax.ShapeDtypeStruct((L, D, B), jnp.float32),
        mesh=mesh,
        compiler_params=pltpu.CompilerParams(
            needs_layout_passes=False, use_tc_tiling_on_sc=True
        ),
        scratch_types=[
            pltpu.VMEM((V,), jnp.float32),
            pltpu.VMEM((B,), jnp.int32),
            pltpu.VMEM((B,), jnp.int32),
            pltpu.VMEM((B,), jnp.int32),
            pltpu.VMEM((B,), jnp.int32),
            pltpu.VMEM((B,), jnp.int32),
            pltpu.VMEM((B,), jnp.float32),
            pltpu.VMEM((B,), jnp.float32),
            pltpu.SemaphoreType.DMA,
            pltpu.SemaphoreType.DMA,
            pltpu.SemaphoreType.DMA,
            pltpu.SemaphoreType.DMA,
            pltpu.SemaphoreType.DMA,
            pltpu.SemaphoreType.DMA,
            pltpu.SemaphoreType.DMA,
        ],
    )(xT, tT)
    return jnp.transpose(outT, (2, 1, 0))
